# baseline (speedup 1.0000x reference)
import jax
import jax.numpy as jnp
from jax.experimental import pallas as pl
from jax.experimental.pallas import tpu as pltpu

DIM = 64
NP1 = 65
CP = 72
BLK = 4096


def _an_kernel(xt_ref, tt_ref, o_ref):
    xt = xt_ref[...]
    tt = tt_ref[...]

    xpt = jnp.dot(tt, xt, preferred_element_type=jnp.float32)
    f = jnp.round(xpt)
    delta = xpt - f
    dsum = jnp.sum(f, axis=0, keepdims=True)

    bits = jax.lax.bitcast_convert_type(delta[:NP1, :], jnp.int32)
    keys = bits ^ jax.lax.shift_right_logical(
        jax.lax.shift_right_arithmetic(bits, 31), 1)
    kmain = keys[:DIM, :]
    kmainp1 = kmain + 1
    isub = jax.lax.broadcasted_iota(jnp.int32, (DIM, BLK), 0)
    rank = jnp.zeros((DIM, BLK), jnp.float32)
    s64 = jnp.zeros((DIM, BLK), jnp.float32)
    for j in range(NP1):
        aj = keys[j:j + 1, :]
        c = aj < jnp.where(isub > j, kmainp1, kmain)
        cf = jnp.where(c, 1.0, 0.0)
        rank = rank + cf
        if j == DIM:
            s64 = cf
    rank64 = DIM - jnp.sum(s64, axis=0, keepdims=True)

    dec = jnp.where((dsum > 0) & (rank < dsum), -1.0, 0.0)
    inc = jnp.where((dsum < 0) & (rank >= NP1 + dsum), 1.0, 0.0)
    fq_main = f[:DIM, :] + dec + inc
    dec64 = jnp.where((dsum > 0) & (rank64 < dsum), -1.0, 0.0)
    inc64 = jnp.where((dsum < 0) & (rank64 >= NP1 + dsum), 1.0, 0.0)
    fq64 = f[DIM:DIM + 1, :] + dec64 + inc64
    isub8 = jax.lax.broadcasted_iota(jnp.int32, (8, BLK), 0)
    tail = jnp.where(isub8 == 0, fq64, 0.0)
    fq = jnp.concatenate([fq_main, tail], axis=0)

    o_ref[...] = jax.lax.dot_general(
        fq, tt, (((0,), (0,)), ((), ())),
        preferred_element_type=jnp.float32)


def _tr_kernel(x_ref, o_ref):
    o_ref[...] = x_ref[...].T


def _transpose(x, n, dim, tb=2048):
    return pl.pallas_call(
        _tr_kernel,
        grid=(n // tb,),
        in_specs=[pl.BlockSpec((tb, dim), lambda i: (i, 0))],
        out_specs=pl.BlockSpec((dim, tb), lambda i: (0, i)),
        out_shape=jax.ShapeDtypeStruct((dim, n), jnp.float32),
    )(x)


def kernel(x, transform):
    n, dim = x.shape
    xt = _transpose(x, n, dim)
    tt_pad = jnp.zeros((CP, dim), jnp.float32).at[:NP1, :].set(transform.T)
    grid = (n // BLK,)
    return pl.pallas_call(
        _an_kernel,
        grid=grid,
        in_specs=[
            pl.BlockSpec((dim, BLK), lambda i: (0, i)),
            pl.BlockSpec((CP, dim), lambda i: (0, 0)),
        ],
        out_specs=pl.BlockSpec((BLK, dim), lambda i: (i, 0)),
        out_shape=jax.ShapeDtypeStruct((n, dim), jnp.float32),
        compiler_params=pltpu.CompilerParams(
            dimension_semantics=("parallel",)),
    )(xt, tt_pad)


# speedup vs baseline: 1.3598x; 1.3598x over previous
import jax
import jax.numpy as jnp
from jax.experimental import pallas as pl
from jax.experimental.pallas import tpu as pltpu

DIM = 64
NP1 = 65
CP = 72
BLK = 4096


def _an_kernel(xt_ref, tt_ref, o_ref):
    xt = xt_ref[...]
    tt = tt_ref[...]

    xpt = jnp.dot(tt, xt, preferred_element_type=jnp.float32)
    f = jnp.round(xpt)
    delta = xpt - f
    dsum = jnp.sum(f, axis=0, keepdims=True)

    bits = jax.lax.bitcast_convert_type(delta[:NP1, :], jnp.int32)
    keys = bits ^ jax.lax.shift_right_logical(
        jax.lax.shift_right_arithmetic(bits, 31), 1)
    kmain = keys[:DIM, :]
    kmainp1 = kmain + 1
    isub = jax.lax.broadcasted_iota(jnp.int32, (DIM, BLK), 0)
    rank = jnp.zeros((DIM, BLK), jnp.float32)
    s64 = jnp.zeros((DIM, BLK), jnp.float32)
    for j in range(NP1):
        aj = keys[j:j + 1, :]
        c = aj < jnp.where(isub > j, kmainp1, kmain)
        cf = jnp.where(c, 1.0, 0.0)
        rank = rank + cf
        if j == DIM:
            s64 = cf
    rank64 = DIM - jnp.sum(s64, axis=0, keepdims=True)

    dec = jnp.where((dsum > 0) & (rank < dsum), -1.0, 0.0)
    inc = jnp.where((dsum < 0) & (rank >= NP1 + dsum), 1.0, 0.0)
    fq_main = f[:DIM, :] + dec + inc
    dec64 = jnp.where((dsum > 0) & (rank64 < dsum), -1.0, 0.0)
    inc64 = jnp.where((dsum < 0) & (rank64 >= NP1 + dsum), 1.0, 0.0)
    fq64 = f[DIM:DIM + 1, :] + dec64 + inc64
    isub8 = jax.lax.broadcasted_iota(jnp.int32, (8, BLK), 0)
    tail = jnp.where(isub8 == 0, fq64, 0.0)
    fq = jnp.concatenate([fq_main, tail], axis=0)

    o_ref[...] = jax.lax.dot_general(
        fq, tt, (((0,), (0,)), ((), ())),
        preferred_element_type=jnp.float32)


def kernel(x, transform):
    n, dim = x.shape
    xt = x.T
    tt_pad = jnp.zeros((CP, dim), jnp.float32).at[:NP1, :].set(transform.T)
    grid = (n // BLK,)
    return pl.pallas_call(
        _an_kernel,
        grid=grid,
        in_specs=[
            pl.BlockSpec((dim, BLK), lambda i: (0, i)),
            pl.BlockSpec((CP, dim), lambda i: (0, 0)),
        ],
        out_specs=pl.BlockSpec((BLK, dim), lambda i: (i, 0)),
        out_shape=jax.ShapeDtypeStruct((n, dim), jnp.float32),
        compiler_params=pltpu.CompilerParams(
            dimension_semantics=("parallel",)),
    )(xt, tt_pad)
